# baseline (speedup 1.0000x reference)
import functools

import jax
import jax.numpy as jnp
from jax import lax
from jax.experimental import pallas as pl
from jax.experimental.pallas import tpu as pltpu
from jax.experimental.pallas import tpu_sc as plsc

N_ROWS = 262144
D = 64
N_IDX = 1048576

NC = 2
NS = 16
L = 16

HALF = N_ROWS // NC
P = 8
CH = HALF // P
ACC_ROWS = CH
SEG = N_IDX // NS
SB = 2048
NBLK = SEG // SB
B = 128
PAIR = 2 * B
CAPB = 3840
THRESH = CAPB - SB
CAP_ALLOC = CAPB + PAIR + L
JUNKPOS = CAP_ALLOC - L


def _body(dst_h, src_h, i0_h, i1_h, w_h, out_h,
          i0b, i1b, wb, i0c, i1c, wc, i0d, i1d, rows, acc):
  c = lax.axis_index("c")
  s = lax.axis_index("s")
  seg0 = s * SEG
  lanes = lax.iota(jnp.int32, L)

  def stage_blk(blk, st):
    pltpu.sync_copy(i0_h.at[pl.ds(seg0 + blk * SB, SB)], i0b.at[st])
    pltpu.sync_copy(i1_h.at[pl.ds(seg0 + blk * SB, SB)], i1b.at[st])
    pltpu.sync_copy(w_h.at[pl.ds(seg0 + blk * SB, SB)], wb.at[st])

  def copy_idx(b):
    for t in range(B // L):
      i0d[pl.ds(t * L, L)] = i0c[pl.ds(b * B + t * L, L)]
      i1d[pl.ds(t * L, L)] = i1c[pl.ds(b * B + t * L, L)]

  def mrow_loop(b):
    def mrow(j, carry2):
      ws = plsc.load_gather(wc, [jnp.full((L,), b * B + j, jnp.int32)])
      for k in range(D // L):
        rows[j, pl.ds(k * L, L)] = rows[j, pl.ds(k * L, L)] * ws
      return carry2
    lax.fori_loop(0, B, mrow, 0)

  def drain(off_v, bad):
    zi = jnp.zeros((L,), jnp.int32)
    zf = jnp.zeros((L,), jnp.float32)
    for t in range(PAIR // L):
      pos = off_v + (t * L) + lanes
      plsc.store_scatter(i1c, [pos], zi)
      plsc.store_scatter(i0c, [pos], zi)
      plsc.store_scatter(wc, [pos], zf)
    off = jnp.max(off_v)
    nb = 2 * ((off + PAIR - 1) // PAIR)

    def fire(b, carry):
      copy_idx(b)
      pltpu.sync_copy(src_h.at[i0d], rows)
      mrow_loop(b)
      pltpu.sync_copy(rows, acc.at[i1d], add=True)
      return carry

    lax.fori_loop(0, nb, fire, 0)

  def make_scan(st):
    def scan_it_inner(i, carry2, ch_rows, base):
      off_v, bad_v = carry2
      v1 = i1b[st, pl.ds(i * L, L)]
      loc = v1 - base
      m = (loc >= 0) & (loc < ch_rows)
      cs = plsc.cumsum(m.astype(jnp.int32))
      pos = jnp.where(m, off_v + cs - 1, JUNKPOS + lanes)
      plsc.store_scatter(i1c, [pos], loc)
      plsc.store_scatter(i0c, [pos], i0b[st, pl.ds(i * L, L)])
      vw = wb[st, pl.ds(i * L, L)]
      plsc.store_scatter(wc, [pos], vw)
      bad_v = jnp.maximum(bad_v, jnp.where(m & (vw != 1.0), 1, 0))
      return (off_v + plsc.all_reduce_population_count(m), bad_v)
    return scan_it_inner

  def maybe_drain(carry):
    off_v, bad_v = carry
    drain(off_v, jnp.max(bad_v))
    zero_v = jnp.zeros((L,), jnp.int32)
    return (zero_v, zero_v)

  def do_pass(pbase, ch_rows, my_rows):
    base = c * HALF + pbase
    r0 = s * my_rows
    pltpu.sync_copy(dst_h.at[pl.ds(base + r0, my_rows)],
                    acc.at[pl.ds(r0, my_rows)])
    plsc.subcore_barrier()

    def blk_pair(q, carry):
      blk0 = 2 * q
      blk1 = blk0 + 1
      stage_blk(blk0, 0)
      scan0 = make_scan(0)
      carry = lax.fori_loop(
          0, SB // L, lambda i, cc: scan0(i, cc, ch_rows, base), carry)
      stage_blk(blk1, 1)
      scan1 = make_scan(1)
      carry = lax.fori_loop(
          0, SB // L, lambda i, cc: scan1(i, cc, ch_rows, base), carry)
      carry = maybe_drain(carry)
      return carry

    zero_v = jnp.zeros((L,), jnp.int32)
    off_v, bad_v = lax.fori_loop(0, NBLK // 2, blk_pair, (zero_v, zero_v))
    drain(off_v, jnp.max(bad_v))

    plsc.subcore_barrier()
    pltpu.sync_copy(acc.at[pl.ds(r0, my_rows)],
                    out_h.at[pl.ds(base + r0, my_rows)])

  def main_passes(p, carry):
    do_pass(p * CH, CH, CH // NS)
    return carry

  lax.fori_loop(0, P, main_passes, 0)


@functools.partial(
    pl.kernel,
    out_type=jax.ShapeDtypeStruct((N_ROWS, D), jnp.float32),
    mesh=plsc.VectorSubcoreMesh(
        core_axis_name="c", subcore_axis_name="s",
        num_cores=NC, num_subcores=NS),
    compiler_params=pltpu.CompilerParams(
        use_tc_tiling_on_sc=False, needs_layout_passes=False),
    scratch_types=[
        pltpu.VMEM((2, SB), jnp.int32),
        pltpu.VMEM((2, SB), jnp.int32),
        pltpu.VMEM((2, SB), jnp.float32),
        pltpu.VMEM((CAP_ALLOC,), jnp.int32),
        pltpu.VMEM((CAP_ALLOC,), jnp.int32),
        pltpu.VMEM((CAP_ALLOC,), jnp.float32),
        pltpu.VMEM((B,), jnp.int32),
        pltpu.VMEM((B,), jnp.int32),
        pltpu.VMEM((B, D), jnp.float32),
        pltpu.VMEM_SHARED((ACC_ROWS, D), jnp.float32),
    ],
)
def _indexed_add(dst_h, src_h, i0_h, i1_h, w_h, out_h, *scratch):
  _body(dst_h, src_h, i0_h, i1_h, w_h, out_h, *scratch)


def kernel(dst, src, index0, index1, weight):
  return _indexed_add(dst, src, index0.astype(jnp.int32),
                      index1.astype(jnp.int32), weight.reshape(-1))


# speedup vs baseline: 1.0257x; 1.0257x over previous
import functools

import jax
import jax.numpy as jnp
from jax import lax
from jax.experimental import pallas as pl
from jax.experimental.pallas import tpu as pltpu
from jax.experimental.pallas import tpu_sc as plsc

N_ROWS = 262144
D = 64
N_IDX = 1048576

NC = 2
NS = 16
L = 16

HALF = N_ROWS // NC
P = 8
CHR = HALF // P
MYR = CHR // NS
SEG = N_IDX // NS
SB = 2048
NBLK = SEG // SB
B = 128
JUNK = SB + 144
CAP = SB + 160


def _body(dst_h, src_h, i0_h, i1_h, w_h, out_h,
          i0_blk, i1_blk, w_blk, i0c, i1c, wc, i0_dma, i1_dma, rows, acc):
  c = lax.axis_index("c")
  s = lax.axis_index("s")
  seg0 = s * SEG

  def do_pass(p, carry):
    base = c * HALF + p * CHR
    r0 = s * MYR
    pltpu.sync_copy(dst_h.at[pl.ds(base + r0, MYR)], acc.at[pl.ds(r0, MYR)])
    plsc.subcore_barrier()

    def do_block(blk, carry2):
      o = seg0 + blk * SB
      pltpu.sync_copy(i1_h.at[pl.ds(o, SB)], i1_blk)
      pltpu.sync_copy(i0_h.at[pl.ds(o, SB)], i0_blk)
      pltpu.sync_copy(w_h.at[pl.ds(o, SB)], w_blk)

      lanes = lax.iota(jnp.int32, L)

      def scan_it(i, off):
        v1 = i1_blk[pl.ds(i * L, L)]
        loc = v1 - base
        m = (loc >= 0) & (loc < CHR)
        mi = m.astype(jnp.int32)
        cs = plsc.cumsum(mi)
        pos = jnp.where(m, off + cs - 1, JUNK + lanes)
        plsc.store_scatter(i1c, [pos], loc)
        plsc.store_scatter(i0c, [pos], i0_blk[pl.ds(i * L, L)])
        plsc.store_scatter(wc, [pos], w_blk[pl.ds(i * L, L)])
        return off + plsc.all_reduce_population_count(m)

      off_v = lax.fori_loop(0, SB // L, scan_it, jnp.zeros((L,), jnp.int32))
      off = jnp.max(off_v)

      zi = jnp.zeros((L,), jnp.int32)
      zf = jnp.zeros((L,), jnp.float32)
      for t in range(9):
        pos = off_v + (t * L) + lanes
        plsc.store_scatter(i1c, [pos], zi)
        plsc.store_scatter(i0c, [pos], zi)
        plsc.store_scatter(wc, [pos], zf)

      nb = (off + B - 1) // B

      def fire(b, carry3):
        for t in range(B // L):
          i0_dma[pl.ds(t * L, L)] = i0c[pl.ds(b * B + t * L, L)]
          i1_dma[pl.ds(t * L, L)] = i1c[pl.ds(b * B + t * L, L)]
        pltpu.sync_copy(src_h.at[i0_dma], rows)

        def mrow(j, carry4):
          ws = plsc.load_gather(wc, [jnp.full((L,), b * B + j, jnp.int32)])
          for k in range(D // L):
            rows[j, pl.ds(k * L, L)] = rows[j, pl.ds(k * L, L)] * ws
          return carry4

        lax.fori_loop(0, B, mrow, 0)
        pltpu.sync_copy(rows, acc.at[i1_dma], add=True)
        return carry3

      lax.fori_loop(0, nb, fire, 0)
      return carry2

    lax.fori_loop(0, NBLK, do_block, 0)

    plsc.subcore_barrier()
    pltpu.sync_copy(acc.at[pl.ds(r0, MYR)], out_h.at[pl.ds(base + r0, MYR)])
    return carry

  lax.fori_loop(0, P, do_pass, 0)


@functools.partial(
    pl.kernel,
    out_type=jax.ShapeDtypeStruct((N_ROWS, D), jnp.float32),
    mesh=plsc.VectorSubcoreMesh(
        core_axis_name="c", subcore_axis_name="s",
        num_cores=NC, num_subcores=NS),
    compiler_params=pltpu.CompilerParams(
        use_tc_tiling_on_sc=False, needs_layout_passes=False),
    scratch_types=[
        pltpu.VMEM((SB,), jnp.int32),
        pltpu.VMEM((SB,), jnp.int32),
        pltpu.VMEM((SB,), jnp.float32),
        pltpu.VMEM((CAP,), jnp.int32),
        pltpu.VMEM((CAP,), jnp.int32),
        pltpu.VMEM((CAP,), jnp.float32),
        pltpu.VMEM((B,), jnp.int32),
        pltpu.VMEM((B,), jnp.int32),
        pltpu.VMEM((B, D), jnp.float32),
        pltpu.VMEM_SHARED((CHR, D), jnp.float32),
    ],
)
def _indexed_add(dst_h, src_h, i0_h, i1_h, w_h, out_h, *scratch):
  _body(dst_h, src_h, i0_h, i1_h, w_h, out_h, *scratch)


def kernel(dst, src, index0, index1, weight):
  return _indexed_add(dst, src, index0.astype(jnp.int32),
                      index1.astype(jnp.int32), weight.reshape(-1))
